# initial kernel scaffold (unmeasured)
import jax
import jax.numpy as jnp
from jax import lax
from jax.experimental import pallas as pl
from jax.experimental.pallas import tpu as pltpu


def kernel(
    x,
):
    def body(*refs):
        pass

    out_shape = jax.ShapeDtypeStruct(..., jnp.float32)
    return pl.pallas_call(body, out_shape=out_shape)(...)


# baseline (device time: 231721 ns/iter reference)
import functools

import jax
import jax.numpy as jnp
from jax import lax
from jax.experimental import pallas as pl
from jax.experimental.pallas import tpu as pltpu


def kernel(x):
    m, n = x.shape
    xb = x.astype(jnp.bfloat16)

    def body(xb_ref, out_ref, recv_ref, send_sem, recv_sem):
        my_x = lax.axis_index("x")
        my_y = lax.axis_index("y")
        my_z = lax.axis_index("z")
        nbr = (my_x, 1 - my_y, my_z)

        barrier_sem = pltpu.get_barrier_semaphore()
        pl.semaphore_signal(
            barrier_sem, inc=1, device_id=nbr,
            device_id_type=pl.DeviceIdType.MESH,
        )
        pl.semaphore_wait(barrier_sem, 1)

        rdma = pltpu.make_async_remote_copy(
            src_ref=xb_ref,
            dst_ref=recv_ref,
            send_sem=send_sem,
            recv_sem=recv_sem,
            device_id=nbr,
            device_id_type=pl.DeviceIdType.MESH,
        )
        rdma.start()
        rdma.wait()

        out_ref[...] = (
            xb_ref[...].astype(jnp.float32) + recv_ref[...].astype(jnp.float32)
        ).astype(jnp.bfloat16)

        @functools.partial(
            pl.run_scoped, exit_sem=pltpu.SemaphoreType.REGULAR
        )
        def _(exit_sem):
            pl.semaphore_signal(
                exit_sem, inc=1, device_id=nbr,
                device_id_type=pl.DeviceIdType.MESH,
            )
            pl.semaphore_wait(exit_sem, 1)

    return pl.pallas_call(
        body,
        out_shape=jax.ShapeDtypeStruct((m, n), jnp.bfloat16),
        in_specs=[pl.BlockSpec(memory_space=pltpu.VMEM)],
        out_specs=pl.BlockSpec(memory_space=pltpu.VMEM),
        scratch_shapes=[
            pltpu.VMEM((m, n), jnp.bfloat16),
            pltpu.SemaphoreType.DMA,
            pltpu.SemaphoreType.DMA,
        ],
        compiler_params=pltpu.CompilerParams(
            collective_id=0, vmem_limit_bytes=100 * 1024 * 1024
        ),
    )(xb)


# device time: 148912 ns/iter; 1.5561x vs baseline; 1.5561x over previous
import jax
import jax.numpy as jnp
from jax import lax
from jax.experimental import pallas as pl
from jax.experimental.pallas import tpu as pltpu

C = 8
HALF = 4096
CH = HALF // C
CH2 = CH // 2


def kernel(x):
    m, n = x.shape
    xb = x.astype(jnp.bfloat16)

    def body(xb_ref, out_ref, recv_ref, sy, ry, sx, rx, sz, rz):
        my_x = lax.axis_index("x")
        my_y = lax.axis_index("y")
        my_z = lax.axis_index("z")
        nbr_y = (my_x, 1 - my_y, my_z)
        nbr_x = (1 - my_x, my_y, my_z)
        nbr_z = (my_x, my_y, 1 - my_z)

        h = (my_x + my_z) % 2
        base_own = h * HALF
        base_opp = (1 - h) * HALF

        barrier_sem = pltpu.get_barrier_semaphore()
        for nbr in (nbr_y, nbr_x, nbr_z):
            pl.semaphore_signal(
                barrier_sem, inc=1, device_id=nbr,
                device_id_type=pl.DeviceIdType.MESH,
            )
        pl.semaphore_wait(barrier_sem, 3)

        y_rdmas = []
        for c in range(C):
            off = base_own + c * CH
            r = pltpu.make_async_remote_copy(
                src_ref=xb_ref.at[pl.ds(off, CH)],
                dst_ref=recv_ref.at[pl.ds(off, CH)],
                send_sem=sy.at[c],
                recv_sem=ry.at[c],
                device_id=nbr_y,
                device_id_type=pl.DeviceIdType.MESH,
            )
            r.start()
            y_rdmas.append(r)

        fwd = []
        for c in range(C):
            y_rdmas[c].wait_recv()
            off = base_own + c * CH
            for sems, nbr, sub in (
                ((sx, rx), nbr_x, 0),
                ((sz, rz), nbr_z, CH2),
            ):
                r = pltpu.make_async_remote_copy(
                    src_ref=recv_ref.at[pl.ds(off + sub, CH2)],
                    dst_ref=recv_ref.at[pl.ds(off + sub, CH2)],
                    send_sem=sems[0].at[c],
                    recv_sem=sems[1].at[c],
                    device_id=nbr,
                    device_id_type=pl.DeviceIdType.MESH,
                )
                r.start()
                fwd.append(r)

        for c in range(C):
            off = base_opp + c * CH
            for sems, nbr, sub in (
                ((sx, rx), nbr_x, 0),
                ((sz, rz), nbr_z, CH2),
            ):
                r = pltpu.make_async_remote_copy(
                    src_ref=recv_ref.at[pl.ds(off + sub, CH2)],
                    dst_ref=recv_ref.at[pl.ds(off + sub, CH2)],
                    send_sem=sems[0].at[c],
                    recv_sem=sems[1].at[c],
                    device_id=nbr,
                    device_id_type=pl.DeviceIdType.MESH,
                )
                r.wait_recv()

        for r in y_rdmas:
            r.wait_send()
        for r in fwd:
            r.wait_send()

        out_ref[...] = (
            xb_ref[...].astype(jnp.float32) + recv_ref[...].astype(jnp.float32)
        ).astype(jnp.bfloat16)

    return pl.pallas_call(
        body,
        out_shape=jax.ShapeDtypeStruct((m, n), jnp.bfloat16),
        in_specs=[pl.BlockSpec(memory_space=pltpu.VMEM)],
        out_specs=pl.BlockSpec(memory_space=pltpu.VMEM),
        scratch_shapes=[
            pltpu.VMEM((m, n), jnp.bfloat16),
            pltpu.SemaphoreType.DMA((C,)),
            pltpu.SemaphoreType.DMA((C,)),
            pltpu.SemaphoreType.DMA((C,)),
            pltpu.SemaphoreType.DMA((C,)),
            pltpu.SemaphoreType.DMA((C,)),
            pltpu.SemaphoreType.DMA((C,)),
        ],
        compiler_params=pltpu.CompilerParams(
            collective_id=0, vmem_limit_bytes=100 * 1024 * 1024
        ),
    )(xb)


# device time: 103958 ns/iter; 2.2290x vs baseline; 1.4324x over previous
import jax
import jax.numpy as jnp
from jax import lax
from jax.experimental import pallas as pl
from jax.experimental.pallas import tpu as pltpu

M = 8192
N = 1024
_MESH = pl.DeviceIdType.MESH


def kernel(x):
    assert x.shape == (M, N), x.shape

    def body(x_hbm, out_hbm, own, recv, stage, ostage,
             in_sems, out_sems, sy, ry, sx, rx, sz, rz):
        my_x = lax.axis_index("x")
        my_y = lax.axis_index("y")
        my_z = lax.axis_index("z")
        nbr_y = (my_x, 1 - my_y, my_z)
        nbr_x = (1 - my_x, my_y, my_z)
        nbr_z = (my_x, my_y, 1 - my_z)

        def i_off(xx, zz):
            return 4096 * xx + zz * (7168 - 8192 * xx)

        I = i_off(my_x, my_z)
        Ix = i_off(1 - my_x, my_z)
        Iz = i_off(my_x, 1 - my_z)
        Id = i_off(1 - my_x, 1 - my_z)
        pair = (my_x + my_z) % 2
        J = 1024 + 4096 * pair
        Jo = 1024 + 4096 * (1 - pair)

        y_list = [(I, 512), (I + 512, 512), (J, 512), (J + 512, 512),
                  (J + 1024, 512), (J + 1536, 256), (J + 1792, 256)]
        x_send = [(I, 512), (I + 512, 512), (J, 512), (J + 512, 512),
                  (Iz + 512, 512)]
        z_send = [(I, 512), (I + 512, 512), (Ix, 512),
                  (J + 1024, 512), (J + 1536, 256), (J + 1792, 256)]
        x_recv = [(Ix, 512), (Ix + 512, 512), (Jo, 512), (Jo + 512, 512),
                  (Id + 512, 512)]
        z_recv = [(Iz, 512), (Iz + 512, 512), (Id, 512),
                  (Jo + 1024, 512), (Jo + 1536, 256), (Jo + 1792, 256)]
        in_list = y_list + [
            (Ix, 512), (Ix + 512, 512), (Iz, 512), (Iz + 512, 512),
            (Id, 512), (Id + 512, 512),
            (Jo, 512), (Jo + 512, 512), (Jo + 1024, 512), (Jo + 1536, 512),
        ]
        n_in = len(in_list)

        barrier_sem = pltpu.get_barrier_semaphore()
        for nbr in (nbr_y, nbr_x, nbr_z):
            pl.semaphore_signal(barrier_sem, inc=1, device_id=nbr,
                                device_id_type=_MESH)
        pl.semaphore_wait(barrier_sem, 3)

        def remote(src, rows, sz_, ssem, rsem, dev):
            return pltpu.make_async_remote_copy(
                src_ref=src.at[pl.ds(rows, sz_)],
                dst_ref=recv.at[pl.ds(rows, sz_)],
                send_sem=ssem, recv_sem=rsem,
                device_id=dev, device_id_type=_MESH,
            )

        y_rd = [remote(own, o, s, sy.at[k], ry.at[k], nbr_y)
                for k, (o, s) in enumerate(y_list)]
        x_rd = [remote(recv, o, s, sx.at[k], rx.at[k], nbr_x)
                for k, (o, s) in enumerate(x_send)]
        z_rd = [remote(recv, o, s, sz.at[k], rz.at[k], nbr_z)
                for k, (o, s) in enumerate(z_send)]
        x_wt = [remote(recv, o, s, sx.at[k], rx.at[k], nbr_x)
                for k, (o, s) in enumerate(x_recv)]
        z_wt = [remote(recv, o, s, sz.at[k], rz.at[k], nbr_z)
                for k, (o, s) in enumerate(z_recv)]

        in_cp = [None] * n_in

        def start_in(k):
            o, s = in_list[k]
            cp = pltpu.make_async_copy(
                x_hbm.at[pl.ds(o, s)], stage.at[k % 2, :s], in_sems.at[k % 2]
            )
            cp.start()
            in_cp[k] = cp

        def cast_in(k):
            o, s = in_list[k]
            in_cp[k].wait()
            own[pl.ds(o, s)] = stage[k % 2, :s].astype(jnp.bfloat16)
            if k + 2 < n_in:
                start_in(k + 2)

        out_cp = [None] * 16
        out_j = [0]

        def emit_out(rows):
            j = out_j[0]
            out_j[0] = j + 1
            if j >= 2:
                out_cp[j - 2].wait()
            ostage[j % 2] = own[pl.ds(rows, 512)] + recv[pl.ds(rows, 512)]
            cp = pltpu.make_async_copy(
                ostage.at[j % 2], out_hbm.at[pl.ds(rows, 512)],
                out_sems.at[j % 2],
            )
            cp.start()
            out_cp[j] = cp

        start_in(0)
        start_in(1)
        for k in range(7):
            cast_in(k)
            y_rd[k].start()

        y_rd[0].wait_recv()
        x_rd[0].start()
        z_rd[0].start()
        cast_in(7)
        emit_out(I)
        y_rd[1].wait_recv()
        x_rd[1].start()
        z_rd[1].start()
        cast_in(8)
        emit_out(I + 512)
        x_wt[0].wait_recv()
        z_rd[2].start()
        cast_in(9)
        emit_out(Ix)
        z_wt[0].wait_recv()
        cast_in(10)
        emit_out(Iz)
        z_wt[1].wait_recv()
        x_rd[4].start()
        cast_in(11)
        emit_out(Iz + 512)
        y_rd[2].wait_recv()
        x_rd[2].start()
        cast_in(12)
        emit_out(J)
        x_wt[1].wait_recv()
        cast_in(13)
        emit_out(Ix + 512)
        y_rd[3].wait_recv()
        x_rd[3].start()
        cast_in(14)
        emit_out(J + 512)
        y_rd[4].wait_recv()
        z_rd[3].start()
        cast_in(15)
        emit_out(J + 1024)
        y_rd[5].wait_recv()
        z_rd[4].start()
        cast_in(16)
        y_rd[6].wait_recv()
        z_rd[5].start()
        emit_out(J + 1536)
        x_wt[2].wait_recv()
        emit_out(Jo)
        x_wt[3].wait_recv()
        emit_out(Jo + 512)
        z_wt[3].wait_recv()
        emit_out(Jo + 1024)
        z_wt[4].wait_recv()
        z_wt[5].wait_recv()
        emit_out(Jo + 1536)
        x_wt[4].wait_recv()
        emit_out(Id + 512)
        z_wt[2].wait_recv()
        emit_out(Id)

        for r in y_rd + x_rd + z_rd:
            r.wait_send()
        out_cp[14].wait()
        out_cp[15].wait()

    return pl.pallas_call(
        body,
        out_shape=jax.ShapeDtypeStruct((M, N), jnp.bfloat16),
        in_specs=[pl.BlockSpec(memory_space=pl.ANY)],
        out_specs=pl.BlockSpec(memory_space=pl.ANY),
        scratch_shapes=[
            pltpu.VMEM((M, N), jnp.bfloat16),
            pltpu.VMEM((M, N), jnp.bfloat16),
            pltpu.VMEM((2, 512, N), jnp.float32),
            pltpu.VMEM((2, 512, N), jnp.bfloat16),
            pltpu.SemaphoreType.DMA((2,)),
            pltpu.SemaphoreType.DMA((2,)),
            pltpu.SemaphoreType.DMA((7,)),
            pltpu.SemaphoreType.DMA((7,)),
            pltpu.SemaphoreType.DMA((5,)),
            pltpu.SemaphoreType.DMA((5,)),
            pltpu.SemaphoreType.DMA((6,)),
            pltpu.SemaphoreType.DMA((6,)),
        ],
        compiler_params=pltpu.CompilerParams(
            collective_id=0, vmem_limit_bytes=100 * 1024 * 1024
        ),
    )(x)
